# baseline (speedup 1.0000x reference)
import functools

import jax
import jax.numpy as jnp
from jax import lax
from jax.experimental import pallas as pl
from jax.experimental.pallas import tpu as pltpu
from jax.experimental.pallas import tpu_sc as plsc

H = 2048
E = 16
N = 16384
TOPK = 2
BT = 512
BP = 2048
NW = 32
C = N // NW
L = 16


def _logits_body(x_ref, w_ref, out_ref):
    out_ref[...] = lax.dot_general(
        w_ref[...], x_ref[...],
        dimension_numbers=(((1,), (1,)), ((), ())),
        preferred_element_type=jnp.float32,
    )


def _logits_tc(x, w):
    return pl.pallas_call(
        _logits_body,
        grid=(N // BT,),
        in_specs=[
            pl.BlockSpec((BT, H), lambda i: (i, 0)),
            pl.BlockSpec((E, H), lambda i: (0, 0)),
        ],
        out_specs=pl.BlockSpec((E, BT), lambda i: (0, i)),
        out_shape=jax.ShapeDtypeStruct((E, N), jnp.float32),
    )(x, w)


def _router_sc_body(lt_hbm, rw_hbm, se_hbm, lt_v, rw_v, se_v):
    wid = lax.axis_index("s") * 2 + lax.axis_index("c")
    base = wid * C
    pltpu.sync_copy(lt_hbm.at[:, pl.ds(base, C)], lt_v)

    neg_inf = jnp.float32(-jnp.inf)

    def step(g, _):
        t0 = g * L
        xs = [lt_v[e, pl.ds(t0, L)] for e in range(E)]
        m1 = functools.reduce(jnp.maximum, xs)
        idx1 = jnp.full((L,), 0, jnp.int32)
        for e in reversed(range(E)):
            idx1 = jnp.where(xs[e] == m1, jnp.int32(e), idx1)
        xs2 = [jnp.where(idx1 == jnp.int32(e), neg_inf, xs[e])
               for e in range(E)]
        m2 = functools.reduce(jnp.maximum, xs2)
        idx2 = jnp.full((L,), 0, jnp.int32)
        for e in reversed(range(E)):
            idx2 = jnp.where(xs2[e] == m2, jnp.int32(e), idx2)
        ex = jnp.exp(m2 - m1)
        denom = jnp.float32(1.0) + ex
        w0 = jnp.float32(1.0) / denom
        w1 = ex / denom
        rw_v[0, pl.ds(t0, L)] = w0
        rw_v[1, pl.ds(t0, L)] = w1
        se_v[0, pl.ds(t0, L)] = idx1
        se_v[1, pl.ds(t0, L)] = idx2
        return _

    lax.fori_loop(0, C // L, step, None)
    pltpu.sync_copy(rw_v, rw_hbm.at[:, pl.ds(base, C)])
    pltpu.sync_copy(se_v, se_hbm.at[:, pl.ds(base, C)])


def _router_sc(logits_t):
    mesh = plsc.VectorSubcoreMesh(core_axis_name="c", subcore_axis_name="s")
    f = pl.kernel(
        _router_sc_body,
        mesh=mesh,
        out_type=[
            jax.ShapeDtypeStruct((TOPK, N), jnp.float32),
            jax.ShapeDtypeStruct((TOPK, N), jnp.int32),
        ],
        scratch_types=[
            pltpu.VMEM((E, C), jnp.float32),
            pltpu.VMEM((TOPK, C), jnp.float32),
            pltpu.VMEM((TOPK, C), jnp.int32),
        ],
    )
    return f(logits_t)


def _pack_body(rwt_ref, set_ref, rw_ref, se_ref):
    rw_ref[...] = rwt_ref[...].T
    se_ref[...] = set_ref[...].T


def _pack_tc(rwt, set_):
    return pl.pallas_call(
        _pack_body,
        grid=(N // BP,),
        in_specs=[
            pl.BlockSpec((TOPK, BP), lambda i: (0, i)),
            pl.BlockSpec((TOPK, BP), lambda i: (0, i)),
        ],
        out_specs=[
            pl.BlockSpec((BP, TOPK), lambda i: (i, 0)),
            pl.BlockSpec((BP, TOPK), lambda i: (i, 0)),
        ],
        out_shape=[
            jax.ShapeDtypeStruct((N, TOPK), jnp.float32),
            jax.ShapeDtypeStruct((N, TOPK), jnp.int32),
        ],
    )(rwt, set_)


def kernel(hidden_states, weight):
    logits_t = _logits_tc(hidden_states, weight)
    rwt, set_ = _router_sc(logits_t)
    routing_weights, selected_experts = _pack_tc(rwt, set_)
    return routing_weights, selected_experts


# speedup vs baseline: 1.2127x; 1.2127x over previous
import functools

import jax
import jax.numpy as jnp
from jax import lax
from jax.experimental import pallas as pl
from jax.experimental.pallas import tpu as pltpu
from jax.experimental.pallas import tpu_sc as plsc

H = 2048
E = 16
N = 16384
TOPK = 2
BT = 512
BP = 2048
NW = 32
C = N // NW
L = 16


def _logits_body(x_ref, w_ref, out_ref):
    out_ref[...] = lax.dot_general(
        w_ref[...], x_ref[...],
        dimension_numbers=(((1,), (1,)), ((), ())),
        preferred_element_type=jnp.float32,
    )


def _logits_tc(x, w):
    return pl.pallas_call(
        _logits_body,
        grid=(N // BT,),
        in_specs=[
            pl.BlockSpec((BT, H), lambda i: (i, 0)),
            pl.BlockSpec((E, H), lambda i: (0, 0)),
        ],
        out_specs=pl.BlockSpec((E, BT), lambda i: (0, i)),
        out_shape=jax.ShapeDtypeStruct((E, N), jnp.float32),
    )(x, w)


def _router_sc_body(lt_hbm, w0_hbm, w1_hbm, i1_hbm, i2_hbm, lt_v, rw_v, se_v):
    wid = lax.axis_index("s") * 2 + lax.axis_index("c")
    base = wid * C
    pltpu.sync_copy(lt_hbm.at[:, pl.ds(base, C)], lt_v)

    neg_inf = jnp.float32(-jnp.inf)

    def step(g, _):
        t0 = g * L
        xs = [lt_v[e, pl.ds(t0, L)] for e in range(E)]
        m1 = functools.reduce(jnp.maximum, xs)
        idx1 = jnp.full((L,), 0, jnp.int32)
        for e in reversed(range(E)):
            idx1 = jnp.where(xs[e] == m1, jnp.int32(e), idx1)
        xs2 = [jnp.where(idx1 == jnp.int32(e), neg_inf, xs[e])
               for e in range(E)]
        m2 = functools.reduce(jnp.maximum, xs2)
        idx2 = jnp.full((L,), 0, jnp.int32)
        for e in reversed(range(E)):
            idx2 = jnp.where(xs2[e] == m2, jnp.int32(e), idx2)
        ex = jnp.exp(m2 - m1)
        denom = jnp.float32(1.0) + ex
        w0 = jnp.float32(1.0) / denom
        w1 = ex / denom
        rw_v[0, pl.ds(t0, L)] = w0
        rw_v[1, pl.ds(t0, L)] = w1
        se_v[0, pl.ds(t0, L)] = idx1
        se_v[1, pl.ds(t0, L)] = idx2
        return _

    lax.fori_loop(0, C // L, step, None)
    pltpu.sync_copy(rw_v.at[0], w0_hbm.at[pl.ds(base, C)])
    pltpu.sync_copy(rw_v.at[1], w1_hbm.at[pl.ds(base, C)])
    pltpu.sync_copy(se_v.at[0], i1_hbm.at[pl.ds(base, C)])
    pltpu.sync_copy(se_v.at[1], i2_hbm.at[pl.ds(base, C)])


def _router_sc(logits_t):
    mesh = plsc.VectorSubcoreMesh(core_axis_name="c", subcore_axis_name="s")
    f = pl.kernel(
        _router_sc_body,
        mesh=mesh,
        out_type=[
            jax.ShapeDtypeStruct((N,), jnp.float32),
            jax.ShapeDtypeStruct((N,), jnp.float32),
            jax.ShapeDtypeStruct((N,), jnp.int32),
            jax.ShapeDtypeStruct((N,), jnp.int32),
        ],
        scratch_types=[
            pltpu.VMEM((E, C), jnp.float32),
            pltpu.VMEM((TOPK, C), jnp.float32),
            pltpu.VMEM((TOPK, C), jnp.int32),
        ],
    )
    return f(logits_t)


def kernel(hidden_states, weight):
    logits_t = _logits_tc(hidden_states, weight)
    w0, w1, i1, i2 = _router_sc(logits_t)
    routing_weights = jnp.stack([w0, w1], axis=-1)
    selected_experts = jnp.stack([i1, i2], axis=-1)
    return routing_weights, selected_experts


# speedup vs baseline: 1.8625x; 1.5358x over previous
import functools

import jax
import jax.numpy as jnp
from jax import lax
from jax.experimental import pallas as pl
from jax.experimental.pallas import tpu as pltpu

H = 2048
E = 16
N = 16384
TOPK = 2
BT = 2048


def _fused_body(x_ref, w_ref, rwt_ref, set_ref):
    logits = lax.dot_general(
        w_ref[...], x_ref[...],
        dimension_numbers=(((1,), (1,)), ((), ())),
        preferred_element_type=jnp.float32,
    )
    neg_inf = jnp.float32(-jnp.inf)
    xs = [lax.slice(logits, (e, 0), (e + 1, BT)) for e in range(E)]
    m1 = functools.reduce(jnp.maximum, xs)
    idx1 = jnp.zeros((1, BT), jnp.int32)
    for e in reversed(range(E)):
        idx1 = jnp.where(xs[e] == m1, jnp.int32(e), idx1)
    xs2 = [jnp.where(idx1 == jnp.int32(e), neg_inf, xs[e]) for e in range(E)]
    m2 = functools.reduce(jnp.maximum, xs2)
    idx2 = jnp.zeros((1, BT), jnp.int32)
    for e in reversed(range(E)):
        idx2 = jnp.where(xs2[e] == m2, jnp.int32(e), idx2)
    ex = jnp.exp(m2 - m1)
    denom = jnp.float32(1.0) + ex
    rwt_ref[0:1, :] = jnp.float32(1.0) / denom
    rwt_ref[1:2, :] = ex / denom
    set_ref[0:1, :] = idx1
    set_ref[1:2, :] = idx2


def _fused_tc(x, w):
    return pl.pallas_call(
        _fused_body,
        grid=(N // BT,),
        in_specs=[
            pl.BlockSpec((BT, H), lambda i: (i, 0)),
            pl.BlockSpec((E, H), lambda i: (0, 0)),
        ],
        out_specs=[
            pl.BlockSpec((TOPK, BT), lambda i: (0, i)),
            pl.BlockSpec((TOPK, BT), lambda i: (0, i)),
        ],
        out_shape=[
            jax.ShapeDtypeStruct((TOPK, N), jnp.float32),
            jax.ShapeDtypeStruct((TOPK, N), jnp.int32),
        ],
    )(x, w)


def kernel(hidden_states, weight):
    rwt, set_ = _fused_tc(hidden_states, weight)
    routing_weights = jnp.stack([rwt[0], rwt[1]], axis=-1)
    selected_experts = jnp.stack([set_[0], set_[1]], axis=-1)
    return routing_weights, selected_experts


# speedup vs baseline: 1.9300x; 1.0363x over previous
import functools

import jax
import jax.numpy as jnp
from jax import lax
from jax.experimental import pallas as pl
from jax.experimental.pallas import tpu as pltpu

H = 2048
E = 16
N = 16384
TOPK = 2
BT = 1024


def _fused_body(x_ref, w_ref, rwt_ref, set_ref):
    logits = lax.dot_general(
        w_ref[...], x_ref[...],
        dimension_numbers=(((1,), (1,)), ((), ())),
        preferred_element_type=jnp.float32,
    )
    neg_inf = jnp.float32(-jnp.inf)
    xs = [lax.slice(logits, (e, 0), (e + 1, BT)) for e in range(E)]
    m1 = functools.reduce(jnp.maximum, xs)
    idx1 = jnp.zeros((1, BT), jnp.int32)
    for e in reversed(range(E)):
        idx1 = jnp.where(xs[e] == m1, jnp.int32(e), idx1)
    xs2 = [jnp.where(idx1 == jnp.int32(e), neg_inf, xs[e]) for e in range(E)]
    m2 = functools.reduce(jnp.maximum, xs2)
    idx2 = jnp.zeros((1, BT), jnp.int32)
    for e in reversed(range(E)):
        idx2 = jnp.where(xs2[e] == m2, jnp.int32(e), idx2)
    ex = jnp.exp(m2 - m1)
    denom = jnp.float32(1.0) + ex
    rwt_ref[0:1, :] = jnp.float32(1.0) / denom
    rwt_ref[1:2, :] = ex / denom
    set_ref[0:1, :] = idx1
    set_ref[1:2, :] = idx2


def _fused_tc(x, w):
    return pl.pallas_call(
        _fused_body,
        grid=(N // BT,),
        in_specs=[
            pl.BlockSpec((BT, H), lambda i: (i, 0)),
            pl.BlockSpec((E, H), lambda i: (0, 0)),
        ],
        out_specs=[
            pl.BlockSpec((TOPK, BT), lambda i: (0, i)),
            pl.BlockSpec((TOPK, BT), lambda i: (0, i)),
        ],
        out_shape=[
            jax.ShapeDtypeStruct((TOPK, N), jnp.float32),
            jax.ShapeDtypeStruct((TOPK, N), jnp.int32),
        ],
    )(x, w)


def kernel(hidden_states, weight):
    rwt, set_ = _fused_tc(hidden_states, weight)
    routing_weights = jnp.stack([rwt[0], rwt[1]], axis=-1)
    selected_experts = jnp.stack([set_[0], set_[1]], axis=-1)
    return routing_weights, selected_experts
